# baseline (speedup 1.0000x reference)
import functools

import jax
import jax.numpy as jnp
from jax import lax
from jax.experimental import pallas as pl
from jax.experimental.pallas import tpu as pltpu
from jax.experimental.pallas import tpu_sc as plsc

N = 100000
D = 128
H = 128
NB = 64
BLK = 2000
GRID = N // BLK

NW = 32
G = N // 16
GW_LO = G // NW
N_HI = G - GW_LO * NW
GW_HI = GW_LO + 1
CH_G = 13
CH_ROWS = CH_G * 16
N_CH = GW_LO // CH_G
WSLICE = GW_LO * 16
WSLICE_HI = GW_HI * 16


def _gate_body(h_ref, w1_ref, b1_ref, w2t_ref, b2_ref, w_ref, m_ref, msc):
    i = pl.program_id(0)
    act = jnp.dot(h_ref[...], w1_ref[...],
                  preferred_element_type=jnp.float32) + b1_ref[...]
    act = act * jax.nn.sigmoid(act)
    w = jnp.sum(act * w2t_ref[...], axis=1, keepdims=True) + b2_ref[0, 0]
    w_ref[...] = w
    bm = jnp.max(w)
    prev = jnp.where(i == 0, -jnp.inf, msc[0, 0])
    msc[0, 0] = jnp.maximum(prev, bm)

    @pl.when(i == GRID - 1)
    def _():
        m_ref[...] = jnp.full((1, 16), msc[0, 0], dtype=jnp.float32)


def _pool_sc_body(h_hbm, w_hbm, b_hbm, m_hbm, num_hbm, den_hbm,
                  hbuf, wbuf, bbuf, ebuf, acc, dacc, mbuf, sem0, sem1):
    cid = lax.axis_index("c")
    sid = lax.axis_index("s")
    wid = sid * 2 + cid
    hi = wid < N_HI
    base_g = jnp.where(hi, wid * GW_HI, N_HI * GW_HI + (wid - N_HI) * GW_LO)
    base_row = base_g * 16

    pltpu.sync_copy(m_hbm, mbuf)
    pltpu.sync_copy(w_hbm.at[pl.ds(base_row, WSLICE)],
                    wbuf.at[pl.ds(0, WSLICE)])
    pltpu.sync_copy(b_hbm.at[pl.ds(base_row, WSLICE)],
                    bbuf.at[pl.ds(0, WSLICE)])

    @pl.when(hi)
    def _():
        pltpu.sync_copy(w_hbm.at[pl.ds(base_row + WSLICE, 16)],
                        wbuf.at[pl.ds(WSLICE, 16)])
        pltpu.sync_copy(b_hbm.at[pl.ds(base_row + WSLICE, 16)],
                        bbuf.at[pl.ds(WSLICE, 16)])

    mvec = mbuf[...]

    def ebody(g, carry):
        ebuf[pl.ds(g * 16, 16)] = jnp.exp(wbuf[pl.ds(g * 16, 16)] - mvec)
        return carry
    lax.fori_loop(0, GW_LO, ebody, 0)

    @pl.when(hi)
    def _():
        ebuf[pl.ds(WSLICE, 16)] = jnp.exp(wbuf[pl.ds(WSLICE, 16)] - mvec)

    z16 = jnp.zeros((16,), jnp.float32)
    ones16 = jnp.ones((16,), jnp.float32)

    def zbody(r, carry):
        for j in range(D // 16):
            acc[r, pl.ds(j * 16, 16)] = z16
        dacc[r, pl.ds(0, 16)] = z16
        return carry
    lax.fori_loop(0, NB, zbody, 0)

    def start(c, slot, sem):
        return pltpu.async_copy(
            h_hbm.at[pl.ds(base_row + c * CH_ROWS, CH_ROWS)],
            hbuf.at[slot], sem)

    def process(slot, ebase_g, ngroups):
        def gbody(g, carry):
            off = (ebase_g + g) * 16
            eg = ebuf[pl.ds(off, 16)]
            bg = bbuf[pl.ds(off, 16)]
            row0 = g * 16
            for r16 in range(16):
                e_r = eg[r16]
                b_r = bg[r16]
                plsc.addupdate(dacc.at[b_r, pl.ds(0, 16)], e_r * ones16)
                for j in range(D // 16):
                    v = hbuf[slot, row0 + r16, pl.ds(j * 16, 16)] * e_r
                    plsc.addupdate(acc.at[b_r, pl.ds(j * 16, 16)], v)
            return carry
        lax.fori_loop(0, ngroups, gbody, 0)

    sems = (sem0, sem1)
    descs = [None, None]
    descs[0] = start(0, 0, sems[0])
    for c in range(N_CH):
        slot = c % 2
        descs[slot].wait()
        if c + 1 < N_CH:
            descs[1 - slot] = start(c + 1, 1 - slot, sems[1 - slot])
        process(slot, c * CH_G, CH_G)

    @pl.when(hi)
    def _():
        pltpu.async_copy(h_hbm.at[pl.ds(base_row + WSLICE, 16)],
                         hbuf.at[0, pl.ds(0, 16)], sem0).wait()
        process(0, GW_LO, 1)

    pltpu.sync_copy(acc, num_hbm.at[wid])
    pltpu.sync_copy(dacc, den_hbm.at[wid])


def _combine_body(num_ref, den_ref, out_ref):
    s = jnp.sum(num_ref[...], axis=0)
    d = jnp.sum(den_ref[...], axis=0)
    dcol = jnp.sum(d, axis=1, keepdims=True) * (1.0 / 16.0)
    out_ref[...] = s / (dcol + 1e-6)


@jax.jit
def kernel(h, batch, W1, b1, W2, b2):
    b1r = b1.reshape(1, H)
    w2t = W2.reshape(1, H)
    b2r = b2.reshape(1, 1)
    bi32 = batch.astype(jnp.int32)

    w, m = pl.pallas_call(
        _gate_body,
        grid=(GRID,),
        in_specs=[
            pl.BlockSpec((BLK, D), lambda i: (i, 0)),
            pl.BlockSpec((D, H), lambda i: (0, 0)),
            pl.BlockSpec((1, H), lambda i: (0, 0)),
            pl.BlockSpec((1, H), lambda i: (0, 0)),
            pl.BlockSpec((1, 1), lambda i: (0, 0)),
        ],
        out_specs=[
            pl.BlockSpec((BLK, 1), lambda i: (i, 0)),
            pl.BlockSpec((1, 16), lambda i: (0, 0)),
        ],
        out_shape=[
            jax.ShapeDtypeStruct((N, 1), jnp.float32),
            jax.ShapeDtypeStruct((1, 16), jnp.float32),
        ],
        scratch_shapes=[pltpu.SMEM((1, 1), jnp.float32)],
    )(h, W1, b1r, w2t, b2r)

    pool = pl.kernel(
        _pool_sc_body,
        out_type=[
            jax.ShapeDtypeStruct((NW, NB, D), jnp.float32),
            jax.ShapeDtypeStruct((NW, NB, 16), jnp.float32),
        ],
        mesh=plsc.VectorSubcoreMesh(core_axis_name="c", subcore_axis_name="s"),
        scratch_types=[
            pltpu.VMEM((2, CH_ROWS, D), jnp.float32),
            pltpu.VMEM((WSLICE_HI,), jnp.float32),
            pltpu.VMEM((WSLICE_HI,), jnp.int32),
            pltpu.VMEM((WSLICE_HI,), jnp.float32),
            pltpu.VMEM((NB, D), jnp.float32),
            pltpu.VMEM((NB, 16), jnp.float32),
            pltpu.VMEM((16,), jnp.float32),
            pltpu.SemaphoreType.DMA,
            pltpu.SemaphoreType.DMA,
        ],
    )
    num_p, den_p = pool(h, w.reshape(N), bi32, m.reshape(16))

    out = pl.pallas_call(
        _combine_body,
        in_specs=[
            pl.BlockSpec((NW, NB, D), lambda: (0, 0, 0)),
            pl.BlockSpec((NW, NB, 16), lambda: (0, 0, 0)),
        ],
        out_specs=pl.BlockSpec((NB, D), lambda: (0, 0)),
        out_shape=jax.ShapeDtypeStruct((NB, D), jnp.float32),
    )(num_p, den_p)
    return out
